# Initial kernel scaffold; baseline (speedup 1.0000x reference)
import jax
import jax.numpy as jnp
from jax.experimental import pallas as pl


def kernel(loc, demand, depot, W_init, b_init, W_ne1, b_ne1, W_f1, b_f1, W_ne2, b_ne2, W_f2, b_f2, W_dep, b_dep):
    raise NotImplementedError("write your pallas kernel here")


# speedup vs baseline: 11.2070x; 11.2070x over previous
import functools

import jax
import jax.numpy as jnp
from jax.experimental import pallas as pl
from jax.experimental.pallas import tpu as pltpu
from jax.experimental.pallas import tpu_sc as plsc

B, N, D, K = 2, 2048, 128, 6
R = 256
NB = N // R
NC, NS = 2, 16
NW = NC * NS
GPW = B * N * K // NW
KC = GPW // 128


def _leaky(v):
    return jnp.where(v >= 0, v, 0.01 * v)


def _prep_body(x_ref, xt_ref, wi_ref, bi_ref, wne1_ref,
               f0_ref, t0_ref, nbr_ref):
    xb = x_ref[0]
    f0 = (xb[:, 0:1] * wi_ref[0:1, :]
          + xb[:, 1:2] * wi_ref[1:2, :]
          + xb[:, 2:3] * wi_ref[2:3, :]
          + bi_ref[0:1, :])
    f0_ref[0] = f0
    t0_ref[0] = jnp.dot(f0, wne1_ref[...],
                        preferred_element_type=jnp.float32)

    xf = xt_ref[0]
    d2 = jnp.sqrt((xb[:, 0:1] - xf[0:1, :]) ** 2
                  + (xb[:, 1:2] - xf[1:2, :]) ** 2
                  + (xb[:, 2:3] - xf[2:3, :]) ** 2)
    iota = jax.lax.broadcasted_iota(jnp.int32, (1, N), 1)
    for t in range(K):
        dmin = jnp.min(d2, axis=1, keepdims=True)
        am = jnp.min(jnp.where(d2 <= dmin, iota, N),
                     axis=1, keepdims=True)
        nbr_ref[0, :, t:t + 1] = am
        d2 = jnp.where(iota == am, jnp.inf, d2)


def _prep_call(x, xt, wi, bi2, wne1):
    return pl.pallas_call(
        _prep_body,
        grid=(B, NB),
        in_specs=[
            pl.BlockSpec((1, R, 3), lambda b, i: (b, i, 0)),
            pl.BlockSpec((1, 3, N), lambda b, i: (b, 0, 0)),
            pl.BlockSpec((3, D), lambda b, i: (0, 0)),
            pl.BlockSpec((1, D), lambda b, i: (0, 0)),
            pl.BlockSpec((D, D), lambda b, i: (0, 0)),
        ],
        out_specs=[
            pl.BlockSpec((1, R, D), lambda b, i: (b, i, 0)),
            pl.BlockSpec((1, R, D), lambda b, i: (b, i, 0)),
            pl.BlockSpec((1, R, K), lambda b, i: (b, i, 0)),
        ],
        out_shape=[
            jax.ShapeDtypeStruct((B, N, D), jnp.float32),
            jax.ShapeDtypeStruct((B, N, D), jnp.float32),
            jax.ShapeDtypeStruct((B, N, K), jnp.int32),
        ],
    )(x, xt, wi, bi2, wne1)


def _sc_gather(table, idx3):
    mesh = plsc.VectorSubcoreMesh(core_axis_name="c", subcore_axis_name="s")

    @functools.partial(
        pl.kernel,
        out_type=jax.ShapeDtypeStruct((NW, KC, 128, D), jnp.float32),
        mesh=mesh,
        scratch_types=[
            pltpu.VMEM((KC, 128), jnp.int32),
            pltpu.VMEM((KC, 128, D), jnp.float32),
            pltpu.SemaphoreType.DMA,
        ],
    )
    def gk(table_hbm, idx_hbm, out_hbm, idx_v, rows_v, sem):
        wid = jax.lax.axis_index("s") * NC + jax.lax.axis_index("c")
        pltpu.sync_copy(idx_hbm.at[wid], idx_v)
        copies = [
            pltpu.async_copy(table_hbm.at[idx_v.at[c]], rows_v.at[c], sem)
            for c in range(KC)
        ]
        for cp in copies:
            cp.wait()
        pltpu.sync_copy(rows_v, out_hbm.at[wid])

    return gk(table, idx3)


def _layer_body(g_ref, t_ref, f_ref, bne_ref, wf_ref, bf_ref, wnx_ref,
                fn_ref, tn_ref):
    c = bne_ref[0:1, :] - t_ref[0]
    s = f_ref[0]
    for j in range(K):
        v = g_ref[0][:, j * D:(j + 1) * D] + c
        s = s + _leaky(v)
    fn = jnp.dot(s, wf_ref[...], preferred_element_type=jnp.float32) \
        + 7.0 * bf_ref[0:1, :]
    fn_ref[0] = fn
    tn_ref[0] = jnp.dot(fn, wnx_ref[...],
                        preferred_element_type=jnp.float32)


def _layer_call(g, t, f, bne2d, wf, bf2d, wnx):
    return pl.pallas_call(
        _layer_body,
        grid=(B, NB),
        in_specs=[
            pl.BlockSpec((1, R, K * D), lambda b, i: (b, i, 0)),
            pl.BlockSpec((1, R, D), lambda b, i: (b, i, 0)),
            pl.BlockSpec((1, R, D), lambda b, i: (b, i, 0)),
            pl.BlockSpec((1, D), lambda b, i: (0, 0)),
            pl.BlockSpec((D, D), lambda b, i: (0, 0)),
            pl.BlockSpec((1, D), lambda b, i: (0, 0)),
            pl.BlockSpec((D, D), lambda b, i: (0, 0)),
        ],
        out_specs=[
            pl.BlockSpec((1, R, D), lambda b, i: (b, i, 0)),
            pl.BlockSpec((1, R, D), lambda b, i: (b, i, 0)),
        ],
        out_shape=[
            jax.ShapeDtypeStruct((B, N, D), jnp.float32),
            jax.ShapeDtypeStruct((B, N, D), jnp.float32),
        ],
    )(g, t, f, bne2d, wf, bf2d, wnx)


def _final_body(g_ref, t_ref, f_ref, bne_ref, wf_ref, bf_ref,
                dp_ref, wdp_ref, bdp_ref,
                h_ref, dep_ref, msum_ref):
    b = pl.program_id(0)
    i = pl.program_id(1)
    c = bne_ref[0:1, :] - t_ref[0]
    s = f_ref[0]
    for j in range(K):
        v = g_ref[0][:, j * D:(j + 1) * D] + c
        s = s + _leaky(v)
    f2 = jnp.dot(s, wf_ref[...], preferred_element_type=jnp.float32) \
        + 7.0 * bf_ref[0:1, :]
    hb = _leaky(f2)
    h_ref[0] = hb
    cs = jnp.sum(hb, axis=0, keepdims=True)

    @pl.when(i == 0)
    def _init():
        row = dp_ref[pl.ds(b, 1), :]
        dep = _leaky(jnp.dot(row, wdp_ref[...],
                             preferred_element_type=jnp.float32)
                     + bdp_ref[0:1, :])
        dep_ref[0] = dep
        msum_ref[0] = cs

    @pl.when(i > 0)
    def _acc():
        msum_ref[0] = msum_ref[0] + cs

    @pl.when(i == NB - 1)
    def _fin():
        msum_ref[0] = (msum_ref[0] + dep_ref[0]) / jnp.float32(N + 1)


def _final_call(g, t, f, bne2d, wf, bf2d, dpad, wdpad, bdp2d):
    return pl.pallas_call(
        _final_body,
        grid=(B, NB),
        in_specs=[
            pl.BlockSpec((1, R, K * D), lambda b, i: (b, i, 0)),
            pl.BlockSpec((1, R, D), lambda b, i: (b, i, 0)),
            pl.BlockSpec((1, R, D), lambda b, i: (b, i, 0)),
            pl.BlockSpec((1, D), lambda b, i: (0, 0)),
            pl.BlockSpec((D, D), lambda b, i: (0, 0)),
            pl.BlockSpec((1, D), lambda b, i: (0, 0)),
            pl.BlockSpec((B, 8), lambda b, i: (0, 0)),
            pl.BlockSpec((8, D), lambda b, i: (0, 0)),
            pl.BlockSpec((1, D), lambda b, i: (0, 0)),
        ],
        out_specs=[
            pl.BlockSpec((1, R, D), lambda b, i: (b, i, 0)),
            pl.BlockSpec((1, 1, D), lambda b, i: (b, 0, 0)),
            pl.BlockSpec((1, 1, D), lambda b, i: (b, 0, 0)),
        ],
        out_shape=[
            jax.ShapeDtypeStruct((B, N, D), jnp.float32),
            jax.ShapeDtypeStruct((B, 1, D), jnp.float32),
            jax.ShapeDtypeStruct((B, 1, D), jnp.float32),
        ],
    )(g, t, f, bne2d, wf, bf2d, dpad, wdpad, bdp2d)


def kernel(loc, demand, depot, W_init, b_init, W_ne1, b_ne1, W_f1, b_f1,
           W_ne2, b_ne2, W_f2, b_f2, W_dep, b_dep):
    x = jnp.concatenate([loc, demand[:, :, None]], axis=2)
    xt = jnp.swapaxes(x, 1, 2)
    f0, t0, nbr = _prep_call(x, xt, W_init, b_init[None, :], W_ne1)

    idx3 = nbr.reshape(NW, KC, 128)
    g1 = _sc_gather(t0[0], idx3).reshape(B, N, K * D)
    f1, t1 = _layer_call(g1, t0, f0, b_ne1[None, :], W_f1,
                         b_f1[None, :], W_ne2)

    g2 = _sc_gather(t1[0], idx3).reshape(B, N, K * D)
    dpad = jnp.concatenate([depot, jnp.zeros((B, 6), jnp.float32)], axis=1)
    wdpad = jnp.concatenate([W_dep, jnp.zeros((6, D), jnp.float32)], axis=0)
    h_body, dep, mean = _final_call(g2, t1, f1, b_ne2[None, :], W_f2,
                                    b_f2[None, :], dpad, wdpad,
                                    b_dep[None, :])
    h = jnp.concatenate([dep, h_body], axis=1)
    return (h, mean[:, 0, :])


# speedup vs baseline: 15.4530x; 1.3789x over previous
import functools

import jax
import jax.numpy as jnp
from jax.experimental import pallas as pl
from jax.experimental.pallas import tpu as pltpu
from jax.experimental.pallas import tpu_sc as plsc

B, N, D, K = 2, 2048, 128, 6
R = 256
NB = N // R
NC, NS = 2, 16
NW = NC * NS
GPW = B * N * K // NW
KC = GPW // 128


def _leaky(v):
    return jnp.where(v >= 0, v, 0.01 * v)


def _prep_body(x_ref, xt_ref, wi_ref, bi_ref, wne1_ref,
               f0_ref, t0_ref, nbr_ref):
    xb = x_ref[0]
    f0 = (xb[:, 0:1] * wi_ref[0:1, :]
          + xb[:, 1:2] * wi_ref[1:2, :]
          + xb[:, 2:3] * wi_ref[2:3, :]
          + bi_ref[0:1, :])
    f0_ref[0] = f0
    t0_ref[0] = jnp.dot(f0, wne1_ref[...],
                        preferred_element_type=jnp.float32)

    xf = xt_ref[0]
    d2 = ((xb[:, 0:1] - xf[0:1, :]) ** 2
          + (xb[:, 1:2] - xf[1:2, :]) ** 2
          + (xb[:, 2:3] - xf[2:3, :]) ** 2)
    iota = jax.lax.broadcasted_iota(jnp.int32, (1, N), 1)
    key = (jax.lax.bitcast_convert_type(d2, jnp.int32)
           & jnp.int32(~0x7FF)) | iota
    imax = jnp.int32(0x7FFFFFFF)
    for t in range(K):
        kmin = jnp.min(key, axis=1, keepdims=True)
        nbr_ref[0, :, t:t + 1] = kmin & jnp.int32(0x7FF)
        key = jnp.where(key <= kmin, imax, key)


def _prep_call(x, xt, wi, bi2, wne1):
    return pl.pallas_call(
        _prep_body,
        grid=(B, NB),
        in_specs=[
            pl.BlockSpec((1, R, 3), lambda b, i: (b, i, 0)),
            pl.BlockSpec((1, 3, N), lambda b, i: (b, 0, 0)),
            pl.BlockSpec((3, D), lambda b, i: (0, 0)),
            pl.BlockSpec((1, D), lambda b, i: (0, 0)),
            pl.BlockSpec((D, D), lambda b, i: (0, 0)),
        ],
        out_specs=[
            pl.BlockSpec((1, R, D), lambda b, i: (b, i, 0)),
            pl.BlockSpec((1, R, D), lambda b, i: (b, i, 0)),
            pl.BlockSpec((1, R, K), lambda b, i: (b, i, 0)),
        ],
        out_shape=[
            jax.ShapeDtypeStruct((B, N, D), jnp.float32),
            jax.ShapeDtypeStruct((B, N, D), jnp.float32),
            jax.ShapeDtypeStruct((B, N, K), jnp.int32),
        ],
    )(x, xt, wi, bi2, wne1)


def _sc_gather(table, idx3):
    mesh = plsc.VectorSubcoreMesh(core_axis_name="c", subcore_axis_name="s")

    @functools.partial(
        pl.kernel,
        out_type=jax.ShapeDtypeStruct((NW, KC, 128, D), jnp.float32),
        mesh=mesh,
        scratch_types=[
            pltpu.VMEM((KC, 128), jnp.int32),
            pltpu.VMEM((KC, 128, D), jnp.float32),
            pltpu.SemaphoreType.DMA,
        ],
    )
    def gk(table_hbm, idx_hbm, out_hbm, idx_v, rows_v, sem):
        wid = jax.lax.axis_index("s") * NC + jax.lax.axis_index("c")
        pltpu.sync_copy(idx_hbm.at[wid], idx_v)
        copies = [
            pltpu.async_copy(table_hbm.at[idx_v.at[c]], rows_v.at[c], sem)
            for c in range(KC)
        ]
        for cp in copies:
            cp.wait()
        pltpu.sync_copy(rows_v, out_hbm.at[wid])

    return gk(table, idx3)


def _gspec(j):
    return pl.BlockSpec((1, R, D), lambda b, i, j=j: (j, b * NB + i, 0))


def _layer_body(g0, g1, g2, g3, g4, g5, t_ref, f_ref, bne_ref, wf_ref,
                bf_ref, wnx_ref, fn_ref, tn_ref):
    c = bne_ref[0:1, :] - t_ref[0]
    s = f_ref[0]
    for g_ref in (g0, g1, g2, g3, g4, g5):
        s = s + _leaky(g_ref[0] + c)
    fn = jnp.dot(s, wf_ref[...], preferred_element_type=jnp.float32) \
        + 7.0 * bf_ref[0:1, :]
    fn_ref[0] = fn
    tn_ref[0] = jnp.dot(fn, wnx_ref[...],
                        preferred_element_type=jnp.float32)


def _layer_call(g, t, f, bne2d, wf, bf2d, wnx):
    return pl.pallas_call(
        _layer_body,
        grid=(B, NB),
        in_specs=[_gspec(j) for j in range(K)] + [
            pl.BlockSpec((1, R, D), lambda b, i: (b, i, 0)),
            pl.BlockSpec((1, R, D), lambda b, i: (b, i, 0)),
            pl.BlockSpec((1, D), lambda b, i: (0, 0)),
            pl.BlockSpec((D, D), lambda b, i: (0, 0)),
            pl.BlockSpec((1, D), lambda b, i: (0, 0)),
            pl.BlockSpec((D, D), lambda b, i: (0, 0)),
        ],
        out_specs=[
            pl.BlockSpec((1, R, D), lambda b, i: (b, i, 0)),
            pl.BlockSpec((1, R, D), lambda b, i: (b, i, 0)),
        ],
        out_shape=[
            jax.ShapeDtypeStruct((B, N, D), jnp.float32),
            jax.ShapeDtypeStruct((B, N, D), jnp.float32),
        ],
    )(g, g, g, g, g, g, t, f, bne2d, wf, bf2d, wnx)


def _final_body(g0, g1, g2, g3, g4, g5, t_ref, f_ref, bne_ref, wf_ref,
                bf_ref, dp_ref, wdp_ref, bdp_ref,
                h_ref, dep_ref, msum_ref):
    b = pl.program_id(0)
    i = pl.program_id(1)
    c = bne_ref[0:1, :] - t_ref[0]
    s = f_ref[0]
    for g_ref in (g0, g1, g2, g3, g4, g5):
        s = s + _leaky(g_ref[0] + c)
    f2 = jnp.dot(s, wf_ref[...], preferred_element_type=jnp.float32) \
        + 7.0 * bf_ref[0:1, :]
    hb = _leaky(f2)
    h_ref[0] = hb
    cs = jnp.sum(hb, axis=0, keepdims=True)

    @pl.when(i == 0)
    def _init():
        row = dp_ref[pl.ds(b, 1), :]
        dep = _leaky(jnp.dot(row, wdp_ref[...],
                             preferred_element_type=jnp.float32)
                     + bdp_ref[0:1, :])
        dep_ref[0] = dep
        msum_ref[0] = cs

    @pl.when(i > 0)
    def _acc():
        msum_ref[0] = msum_ref[0] + cs

    @pl.when(i == NB - 1)
    def _fin():
        msum_ref[0] = (msum_ref[0] + dep_ref[0]) / jnp.float32(N + 1)


def _final_call(g, t, f, bne2d, wf, bf2d, dpad, wdpad, bdp2d):
    return pl.pallas_call(
        _final_body,
        grid=(B, NB),
        in_specs=[_gspec(j) for j in range(K)] + [
            pl.BlockSpec((1, R, D), lambda b, i: (b, i, 0)),
            pl.BlockSpec((1, R, D), lambda b, i: (b, i, 0)),
            pl.BlockSpec((1, D), lambda b, i: (0, 0)),
            pl.BlockSpec((D, D), lambda b, i: (0, 0)),
            pl.BlockSpec((1, D), lambda b, i: (0, 0)),
            pl.BlockSpec((B, 8), lambda b, i: (0, 0)),
            pl.BlockSpec((8, D), lambda b, i: (0, 0)),
            pl.BlockSpec((1, D), lambda b, i: (0, 0)),
        ],
        out_specs=[
            pl.BlockSpec((1, R, D), lambda b, i: (b, i, 0)),
            pl.BlockSpec((1, 1, D), lambda b, i: (b, 0, 0)),
            pl.BlockSpec((1, 1, D), lambda b, i: (b, 0, 0)),
        ],
        out_shape=[
            jax.ShapeDtypeStruct((B, N, D), jnp.float32),
            jax.ShapeDtypeStruct((B, 1, D), jnp.float32),
            jax.ShapeDtypeStruct((B, 1, D), jnp.float32),
        ],
    )(g, g, g, g, g, g, t, f, bne2d, wf, bf2d, dpad, wdpad, bdp2d)


def kernel(loc, demand, depot, W_init, b_init, W_ne1, b_ne1, W_f1, b_f1,
           W_ne2, b_ne2, W_f2, b_f2, W_dep, b_dep):
    x = jnp.concatenate([loc, demand[:, :, None]], axis=2)
    xt = jnp.swapaxes(x, 1, 2)
    f0, t0, nbr = _prep_call(x, xt, W_init, b_init[None, :], W_ne1)

    idx3 = nbr.transpose(2, 0, 1).reshape(NW, KC, 128)
    g1 = _sc_gather(t0[0], idx3).reshape(K, B * N, D)
    f1, t1 = _layer_call(g1, t0, f0, b_ne1[None, :], W_f1,
                         b_f1[None, :], W_ne2)

    g2 = _sc_gather(t1[0], idx3).reshape(K, B * N, D)
    dpad = jnp.concatenate([depot, jnp.zeros((B, 6), jnp.float32)], axis=1)
    wdpad = jnp.concatenate([W_dep, jnp.zeros((6, D), jnp.float32)], axis=0)
    h_body, dep, mean = _final_call(g2, t1, f1, b_ne2[None, :], W_f2,
                                    b_f2[None, :], dpad, wdpad,
                                    b_dep[None, :])
    h = jnp.concatenate([dep, h_body], axis=1)
    return (h, mean[:, 0, :])


# speedup vs baseline: 15.6964x; 1.0157x over previous
import functools

import jax
import jax.numpy as jnp
from jax.experimental import pallas as pl
from jax.experimental.pallas import tpu as pltpu
from jax.experimental.pallas import tpu_sc as plsc

B, N, D, K = 2, 2048, 128, 6
R = 256
NB = N // R
NC, NS = 2, 16
NW = NC * NS
GPW = B * N * K // NW
KC = GPW // 128


def _leaky(v):
    return jnp.where(v >= 0, v, 0.01 * v)


def _prep_body(x_ref, xfull_ref, xt_ref, wi_ref, bi_ref, wne1_ref,
               f0_ref, t0_ref, nbr_ref):
    xb = x_ref[0]
    f0 = (xb[:, 0:1] * wi_ref[0:1, :]
          + xb[:, 1:2] * wi_ref[1:2, :]
          + xb[:, 2:3] * wi_ref[2:3, :]
          + bi_ref[0:1, :])
    f0_ref[0] = f0
    t0_ref[0] = jnp.dot(f0, wne1_ref[...],
                        preferred_element_type=jnp.float32)

    xf = xfull_ref[0]
    xr = xt_ref[0]
    d2 = ((xf[:, 0:1] - xr[0:1, :]) ** 2
          + (xf[:, 1:2] - xr[1:2, :]) ** 2
          + (xf[:, 2:3] - xr[2:3, :]) ** 2)
    iota = jax.lax.broadcasted_iota(jnp.int32, (N, 1), 0)
    key = (jax.lax.bitcast_convert_type(d2, jnp.int32)
           & jnp.int32(~0x7FF)) | iota
    imax = jnp.int32(0x7FFFFFFF)
    for t in range(K):
        kmin = jnp.min(key, axis=0, keepdims=True)
        nbr_ref[0, t:t + 1, :] = kmin & jnp.int32(0x7FF)
        key = jnp.where(key <= kmin, imax, key)


def _prep_call(x, xt, wi, bi2, wne1):
    return pl.pallas_call(
        _prep_body,
        grid=(B, NB),
        in_specs=[
            pl.BlockSpec((1, R, 3), lambda b, i: (b, i, 0)),
            pl.BlockSpec((1, N, 3), lambda b, i: (b, 0, 0)),
            pl.BlockSpec((1, 3, R), lambda b, i: (b, 0, i)),
            pl.BlockSpec((3, D), lambda b, i: (0, 0)),
            pl.BlockSpec((1, D), lambda b, i: (0, 0)),
            pl.BlockSpec((D, D), lambda b, i: (0, 0)),
        ],
        out_specs=[
            pl.BlockSpec((1, R, D), lambda b, i: (b, i, 0)),
            pl.BlockSpec((1, R, D), lambda b, i: (b, i, 0)),
            pl.BlockSpec((1, K, R), lambda b, i: (b, 0, i)),
        ],
        out_shape=[
            jax.ShapeDtypeStruct((B, N, D), jnp.float32),
            jax.ShapeDtypeStruct((B, N, D), jnp.float32),
            jax.ShapeDtypeStruct((B, K, N), jnp.int32),
        ],
    )(x, x, xt, wi, bi2, wne1)


def _sc_gather(table, idx3):
    mesh = plsc.VectorSubcoreMesh(core_axis_name="c", subcore_axis_name="s")

    @functools.partial(
        pl.kernel,
        out_type=jax.ShapeDtypeStruct((NW, KC, 128, D), jnp.float32),
        mesh=mesh,
        scratch_types=[
            pltpu.VMEM((KC, 128), jnp.int32),
            pltpu.VMEM((KC, 128, D), jnp.float32),
            pltpu.SemaphoreType.DMA,
        ],
    )
    def gk(table_hbm, idx_hbm, out_hbm, idx_v, rows_v, sem):
        wid = jax.lax.axis_index("s") * NC + jax.lax.axis_index("c")
        pltpu.sync_copy(idx_hbm.at[wid], idx_v)
        copies = [
            pltpu.async_copy(table_hbm.at[idx_v.at[c]], rows_v.at[c], sem)
            for c in range(KC)
        ]
        for cp in copies:
            cp.wait()
        pltpu.sync_copy(rows_v, out_hbm.at[wid])

    return gk(table, idx3)


def _gspec(j):
    return pl.BlockSpec((1, 1, R, D), lambda b, i, j=j: (b, j, i, 0))


def _layer_body(g0, g1, g2, g3, g4, g5, t_ref, f_ref, bne_ref, wf_ref,
                bf_ref, wnx_ref, fn_ref, tn_ref):
    c = bne_ref[0:1, :] - t_ref[0]
    s = f_ref[0]
    for g_ref in (g0, g1, g2, g3, g4, g5):
        s = s + _leaky(g_ref[0, 0] + c)
    fn = jnp.dot(s, wf_ref[...], preferred_element_type=jnp.float32) \
        + 7.0 * bf_ref[0:1, :]
    fn_ref[0] = fn
    tn_ref[0] = jnp.dot(fn, wnx_ref[...],
                        preferred_element_type=jnp.float32)


def _layer_call(g, t, f, bne2d, wf, bf2d, wnx):
    return pl.pallas_call(
        _layer_body,
        grid=(B, NB),
        in_specs=[_gspec(j) for j in range(K)] + [
            pl.BlockSpec((1, R, D), lambda b, i: (b, i, 0)),
            pl.BlockSpec((1, R, D), lambda b, i: (b, i, 0)),
            pl.BlockSpec((1, D), lambda b, i: (0, 0)),
            pl.BlockSpec((D, D), lambda b, i: (0, 0)),
            pl.BlockSpec((1, D), lambda b, i: (0, 0)),
            pl.BlockSpec((D, D), lambda b, i: (0, 0)),
        ],
        out_specs=[
            pl.BlockSpec((1, R, D), lambda b, i: (b, i, 0)),
            pl.BlockSpec((1, R, D), lambda b, i: (b, i, 0)),
        ],
        out_shape=[
            jax.ShapeDtypeStruct((B, N, D), jnp.float32),
            jax.ShapeDtypeStruct((B, N, D), jnp.float32),
        ],
    )(g, g, g, g, g, g, t, f, bne2d, wf, bf2d, wnx)


def _final_body(g0, g1, g2, g3, g4, g5, t_ref, f_ref, bne_ref, wf_ref,
                bf_ref, dp_ref, wdp_ref, bdp_ref,
                h_ref, dep_ref, msum_ref):
    b = pl.program_id(0)
    i = pl.program_id(1)
    c = bne_ref[0:1, :] - t_ref[0]
    s = f_ref[0]
    for g_ref in (g0, g1, g2, g3, g4, g5):
        s = s + _leaky(g_ref[0, 0] + c)
    f2 = jnp.dot(s, wf_ref[...], preferred_element_type=jnp.float32) \
        + 7.0 * bf_ref[0:1, :]
    hb = _leaky(f2)
    h_ref[0] = hb
    cs = jnp.sum(hb, axis=0, keepdims=True)

    @pl.when(i == 0)
    def _init():
        row = dp_ref[pl.ds(b, 1), :]
        dep = _leaky(jnp.dot(row, wdp_ref[...],
                             preferred_element_type=jnp.float32)
                     + bdp_ref[0:1, :])
        dep_ref[0] = dep
        msum_ref[0] = cs

    @pl.when(i > 0)
    def _acc():
        msum_ref[0] = msum_ref[0] + cs

    @pl.when(i == NB - 1)
    def _fin():
        msum_ref[0] = (msum_ref[0] + dep_ref[0]) / jnp.float32(N + 1)


def _final_call(g, t, f, bne2d, wf, bf2d, dpad, wdpad, bdp2d):
    return pl.pallas_call(
        _final_body,
        grid=(B, NB),
        in_specs=[_gspec(j) for j in range(K)] + [
            pl.BlockSpec((1, R, D), lambda b, i: (b, i, 0)),
            pl.BlockSpec((1, R, D), lambda b, i: (b, i, 0)),
            pl.BlockSpec((1, D), lambda b, i: (0, 0)),
            pl.BlockSpec((D, D), lambda b, i: (0, 0)),
            pl.BlockSpec((1, D), lambda b, i: (0, 0)),
            pl.BlockSpec((B, 8), lambda b, i: (0, 0)),
            pl.BlockSpec((8, D), lambda b, i: (0, 0)),
            pl.BlockSpec((1, D), lambda b, i: (0, 0)),
        ],
        out_specs=[
            pl.BlockSpec((1, R, D), lambda b, i: (b, i, 0)),
            pl.BlockSpec((1, 1, D), lambda b, i: (b, 0, 0)),
            pl.BlockSpec((1, 1, D), lambda b, i: (b, 0, 0)),
        ],
        out_shape=[
            jax.ShapeDtypeStruct((B, N, D), jnp.float32),
            jax.ShapeDtypeStruct((B, 1, D), jnp.float32),
            jax.ShapeDtypeStruct((B, 1, D), jnp.float32),
        ],
    )(g, g, g, g, g, g, t, f, bne2d, wf, bf2d, dpad, wdpad, bdp2d)


def kernel(loc, demand, depot, W_init, b_init, W_ne1, b_ne1, W_f1, b_f1,
           W_ne2, b_ne2, W_f2, b_f2, W_dep, b_dep):
    x = jnp.concatenate([loc, demand[:, :, None]], axis=2)
    xt = jnp.swapaxes(x, 1, 2)
    f0, t0, nbr = _prep_call(x, xt, W_init, b_init[None, :], W_ne1)

    idx3 = nbr.reshape(NW, KC, 128)
    g1 = _sc_gather(t0[0], idx3).reshape(B, K, N, D)
    f1, t1 = _layer_call(g1, t0, f0, b_ne1[None, :], W_f1,
                         b_f1[None, :], W_ne2)

    g2 = _sc_gather(t1[0], idx3).reshape(B, K, N, D)
    dpad = jnp.concatenate([depot, jnp.zeros((B, 6), jnp.float32)], axis=1)
    wdpad = jnp.concatenate([W_dep, jnp.zeros((6, D), jnp.float32)], axis=0)
    h_body, dep, mean = _final_call(g2, t1, f1, b_ne2[None, :], W_f2,
                                    b_f2[None, :], dpad, wdpad,
                                    b_dep[None, :])
    h = jnp.concatenate([dep, h_body], axis=1)
    return (h, mean[:, 0, :])
